# baseline (speedup 1.0000x reference)
import functools

import jax
import jax.numpy as jnp
from jax import lax
from jax.experimental import pallas as pl
from jax.experimental.pallas import tpu as pltpu
from jax.experimental.pallas import tpu_sc as plsc

_N = 16384
_W = 1024
_K = 10240
_BLK = 2048

_L = 16
_NC = 2
_NS = 16
_NW = _NC * _NS
_NSC = _N - _K
_RPW = _NSC // _NW
_C = 16
_NCHUNK = _RPW // _C
_RU = 8


def _tc_blend(ids_ref, prev_ref, tab_ref, out_ref):
    t0 = tab_ref[0, :]
    t1 = tab_ref[1, :]
    sel = ids_ref[...][:, :1].astype(jnp.float32)
    out_ref[...] = prev_ref[...] + (t0 + sel * (t1 - t0))


def _sc_body(sel_hbm, prev_hbm, tab_hbm, out_hbm,
             sel_v, selx_v, tab_v, buf0_v, buf1_v,
             si0, si1, so0, so1):
    wid = lax.axis_index("s") * _NC + lax.axis_index("c")
    base = wid * _RPW
    pltpu.sync_copy(sel_hbm.at[pl.ds(base, _RPW)], sel_v)
    pltpu.sync_copy(tab_hbm, tab_v)

    def expand_body(g, _):
        v16 = sel_v[pl.ds(g * _L, _L)]
        for k in range(_L):
            selx_v[g * _L + k, :] = jnp.broadcast_to(v16[k], (_L,))
        return ()

    lax.fori_loop(0, _RPW // _L, expand_body, ())

    bufs = (buf0_v, buf1_v)
    in_sems = (si0, si1)
    out_sems = (so0, so1)
    h_in = []
    h_out = []
    for ci in range(_NCHUNK):
        row0 = _K + base + ci * _C
        b = bufs[ci % 2]
        h_in.append(pltpu.make_async_copy(
            prev_hbm.at[pl.ds(row0, _C), :], b, in_sems[ci % 2]))
        h_out.append(pltpu.make_async_copy(
            b, out_hbm.at[pl.ds(base + ci * _C, _C), :], out_sems[ci % 2]))

    h_in[0].start()
    for ci in range(_NCHUNK):
        if ci + 1 < _NCHUNK:
            if ci - 1 >= 0:
                h_out[ci - 1].wait()
            h_in[ci + 1].start()
        h_in[ci].wait()
        buf = bufs[ci % 2]

        def grp_body(g, _, _ci=ci, _buf=buf):
            selvs = [selx_v[_ci * _C + g * _RU + k, :] for k in range(_RU)]

            def col_body(j, _):
                sl = pl.ds(j * _L, _L)
                t0 = tab_v[0, sl]
                d = tab_v[1, sl] - t0
                for k in range(_RU):
                    r = g * _RU + k
                    _buf[r, sl] = _buf[r, sl] + (t0 + selvs[k] * d)
                return ()

            lax.fori_loop(0, _W // _L, col_body, ())
            return ()

        lax.fori_loop(0, _C // _RU, grp_body, ())
        h_out[ci].start()

    h_out[_NCHUNK - 2].wait()
    h_out[_NCHUNK - 1].wait()


@functools.partial(
    pl.kernel,
    out_type=jax.ShapeDtypeStruct((_NSC, _W), jnp.float32),
    mesh=plsc.VectorSubcoreMesh(core_axis_name="c", subcore_axis_name="s"),
    scratch_types=[
        pltpu.VMEM((_RPW,), jnp.float32),
        pltpu.VMEM((_RPW, _L), jnp.float32),
        pltpu.VMEM((2, _W), jnp.float32),
        pltpu.VMEM((_C, _W), jnp.float32),
        pltpu.VMEM((_C, _W), jnp.float32),
        pltpu.SemaphoreType.DMA,
        pltpu.SemaphoreType.DMA,
        pltpu.SemaphoreType.DMA,
        pltpu.SemaphoreType.DMA,
    ],
)
def _sc_blend(*refs):
    _sc_body(*refs)


def kernel(previous_embedding, token_type_ids, token_type_table):
    b, s, w = previous_embedding.shape
    n = b * s
    prev = previous_embedding.reshape(n, w)
    ids_flat = token_type_ids.reshape(n)
    ids_tc = jnp.broadcast_to(
        ids_flat[:_K].reshape(_K, 1).astype(jnp.int8), (_K, 128))
    sel_sc = ids_flat[_K:].astype(jnp.float32)

    out_tc = pl.pallas_call(
        _tc_blend,
        grid=(_K // _BLK,),
        in_specs=[
            pl.BlockSpec((_BLK, 128), lambda i: (i, 0)),
            pl.BlockSpec((_BLK, w), lambda i: (i, 0)),
            pl.BlockSpec((2, w), lambda i: (0, 0)),
        ],
        out_specs=pl.BlockSpec((_BLK, w), lambda i: (i, 0)),
        out_shape=jax.ShapeDtypeStruct((_K, w), jnp.float32),
    )(ids_tc, prev, token_type_table)

    out_sc = _sc_blend(sel_sc, prev, token_type_table)
    out = jnp.concatenate([out_tc, out_sc], axis=0)
    return out.reshape(b, s, w)


# speedup vs baseline: 2.2370x; 2.2370x over previous
import jax
import jax.numpy as jnp
from jax.experimental import pallas as pl

_BLK = 1024


def _blend_kernel(ids_ref, prev_ref, tab_ref, out_ref):
    t0 = tab_ref[0, :]
    t1 = tab_ref[1, :]
    sel = ids_ref[...][:, :1].astype(jnp.float32)
    out_ref[...] = prev_ref[...] + (t0 + sel * (t1 - t0))


def kernel(previous_embedding, token_type_ids, token_type_table):
    b, s, w = previous_embedding.shape
    n = b * s
    prev = previous_embedding.reshape(n, w)
    ids = jnp.broadcast_to(
        token_type_ids.reshape(n, 1).astype(jnp.int8), (n, 128))
    out = pl.pallas_call(
        _blend_kernel,
        grid=(n // _BLK,),
        in_specs=[
            pl.BlockSpec((_BLK, 128), lambda i: (i, 0)),
            pl.BlockSpec((_BLK, w), lambda i: (i, 0)),
            pl.BlockSpec((2, w), lambda i: (0, 0)),
        ],
        out_specs=pl.BlockSpec((_BLK, w), lambda i: (i, 0)),
        out_shape=jax.ShapeDtypeStruct((n, w), jnp.float32),
    )(ids, prev, token_type_table)
    return out.reshape(b, s, w)


# speedup vs baseline: 2.5019x; 1.1184x over previous
import jax
import jax.numpy as jnp
from jax.experimental import pallas as pl

_BLK = 2048


def _blend_kernel(ids_ref, prev_ref, tab_ref, out_ref):
    sel = ids_ref[0, 0, :]
    oh_t = jnp.stack([1.0 - sel, sel], axis=0)
    emb = jax.lax.dot_general(
        oh_t, tab_ref[...], (((0,), (0,)), ((), ())),
        preferred_element_type=jnp.float32)
    out_ref[...] = prev_ref[...] + emb


def kernel(previous_embedding, token_type_ids, token_type_table):
    b, s, w = previous_embedding.shape
    n = b * s
    prev = previous_embedding.reshape(n, w)
    nb = n // _BLK
    ids = token_type_ids.reshape(nb, 1, _BLK).astype(jnp.float32)
    out = pl.pallas_call(
        _blend_kernel,
        grid=(nb,),
        in_specs=[
            pl.BlockSpec((1, 1, _BLK), lambda i: (i, 0, 0)),
            pl.BlockSpec((_BLK, w), lambda i: (i, 0)),
            pl.BlockSpec((2, w), lambda i: (0, 0)),
        ],
        out_specs=pl.BlockSpec((_BLK, w), lambda i: (i, 0)),
        out_shape=jax.ShapeDtypeStruct((n, w), jnp.float32),
    )(ids, prev, token_type_table)
    return out.reshape(b, s, w)
